# baseline (speedup 1.0000x reference)
import functools

import jax
import jax.numpy as jnp
from jax import lax
from jax.experimental import pallas as pl
from jax.experimental.pallas import tpu as pltpu
from jax.experimental.pallas import tpu_sc as plsc

N = 10000
E = 320000
IN_CH = 128
HID = 256
OUT_CH = 128

NC = 2
NS = 16
CH = 128
EROWS = 2560
EPAD = EROWS * CH
NPAD = 10240
ZR = 128
RPT = NPAD // NS

_MESH = dict(core_axis_name="c", subcore_axis_name="s", num_cores=NC,
             num_subcores=NS)


def _fill(ref, nrows, ncols, value):
    vec = jnp.full((16,), value, jnp.float32)

    def body(i, _):
        for t in range(ncols // 16):
            ref[i, pl.ds(t * 16, 16)] = vec
        return 0

    lax.fori_loop(0, nrows, body, 0)


_DROWS = EROWS // (NC * NS)


@functools.cache
def _make_deg_kernel():
    @functools.partial(
        pl.kernel,
        out_type=jax.ShapeDtypeStruct((NC, NPAD, 16), jnp.float32),
        mesh=plsc.VectorSubcoreMesh(**_MESH),
        scratch_types=[
            pltpu.VMEM((_DROWS, CH), jnp.int32),
            pltpu.VMEM((CH, 16), jnp.float32),
            pltpu.VMEM((ZR, 16), jnp.float32),
            pltpu.VMEM_SHARED((NPAD, 16), jnp.float32),
            pltpu.SemaphoreType.DMA,
            pltpu.SemaphoreType.DMA,
        ],
    )
    def deg(dst_hbm, out_hbm, didx, ones, zbuf, acc, sem0, sem1):
        c = lax.axis_index("c")
        s = lax.axis_index("s")
        w32 = c * NS + s
        base = w32 * _DROWS
        n = _DROWS

        slab = pltpu.async_copy(dst_hbm.at[pl.ds(base, _DROWS)], didx, sem0)
        _fill(ones, CH, 16, 1.0)
        _fill(zbuf, ZR, 16, 0.0)
        for j in range(RPT // ZR):
            pltpu.sync_copy(zbuf, acc.at[pl.ds(s * RPT + j * ZR, ZR)])
        slab.wait()
        plsc.subcore_barrier()

        def sadd(jj, sem):
            return pltpu.async_copy(ones, acc.at[didx.at[jj]], sem, add=True)

        sadd(0, sem0)
        sadd(1, sem1)

        def pair(j2, _):
            jj = 2 * j2
            pltpu.make_async_copy(ones, acc.at[didx.at[jj]], sem0).wait()

            @pl.when(jj + 2 < n)
            def _():
                sadd(jj + 2, sem0)

            pltpu.make_async_copy(ones, acc.at[didx.at[jj]], sem1).wait()

            @pl.when(jj + 3 < n)
            def _():
                sadd(jj + 3, sem1)

            return 0

        lax.fori_loop(0, n // 2, pair, 0)
        plsc.subcore_barrier()
        for j in range(RPT // ZR):
            r0 = s * RPT + j * ZR
            pltpu.sync_copy(acc.at[pl.ds(r0, ZR)], zbuf)
            pltpu.sync_copy(zbuf, out_hbm.at[c, pl.ds(r0, ZR)])

    return deg


@functools.cache
def _make_agg_kernel(w, split_channels):
    n = EROWS // NS if split_channels else _DROWS

    @functools.partial(
        pl.kernel,
        out_type=jax.ShapeDtypeStruct((NC, NPAD, w), jnp.float32),
        mesh=plsc.VectorSubcoreMesh(**_MESH),
        scratch_types=[
            pltpu.VMEM((2, CH), jnp.int32),
            pltpu.VMEM((2, CH), jnp.int32),
            pltpu.VMEM((CH, w), jnp.float32),
            pltpu.VMEM((CH, w), jnp.float32),
            pltpu.VMEM((64, w), jnp.float32),
            pltpu.VMEM_SHARED((NPAD, w), jnp.float32),
            pltpu.SemaphoreType.DMA,
            pltpu.SemaphoreType.DMA,
            pltpu.SemaphoreType.DMA,
            pltpu.SemaphoreType.DMA,
        ],
    )
    def agg(y_hbm, src_hbm, dst_hbm, out_hbm, sidx, didx, rows0, rows1,
            zbuf, acc, isem0, isem1, gsem0, gsem1):
        c = lax.axis_index("c")
        s = lax.axis_index("s")
        base = (s if split_channels else c * NS + s) * n

        rows = (rows0, rows1)
        isems = (isem0, isem1)
        gsems = (gsem0, gsem1)

        def idx_start(j, b):
            e0 = (base + j) * CH
            pltpu.async_copy(src_hbm.at[pl.ds(e0, CH)], sidx.at[b], isems[b])
            pltpu.async_copy(dst_hbm.at[pl.ds(e0, CH)], didx.at[b], isems[b])

        def idx_wait(b):
            pltpu.make_async_copy(src_hbm.at[pl.ds(0, CH)], sidx.at[b],
                                  isems[b]).wait()
            pltpu.make_async_copy(dst_hbm.at[pl.ds(0, CH)], didx.at[b],
                                  isems[b]).wait()

        def shift_src(b):
            if split_channels:
                @pl.when(c == 1)
                def _():
                    for t in range(CH // 16):
                        sl = pl.ds(t * 16, 16)
                        sidx[b, sl] = sidx[b, sl] + N

        def gstart(b):
            pltpu.async_copy(y_hbm.at[sidx.at[b]], rows[b], gsems[b])

        def gwait(b):
            pltpu.make_async_copy(y_hbm.at[sidx.at[b]], rows[b],
                                  gsems[b]).wait()

        def scatter(b):
            pltpu.sync_copy(rows[b], acc.at[didx.at[b]], add=True)

        idx_start(0, 0)
        _fill(zbuf, 64, w, 0.0)
        for j in range(RPT // 64):
            pltpu.sync_copy(zbuf, acc.at[pl.ds(s * RPT + j * 64, 64)])
        plsc.subcore_barrier()

        idx_wait(0)
        shift_src(0)
        gstart(0)
        idx_start(1, 1)

        def step(j, b):
            idx_wait(b)
            shift_src(b)
            gstart(b)
            gwait(1 - b)
            scatter(1 - b)

            @pl.when(j + 1 < n)
            def _():
                idx_start(j + 1, 1 - b)

        def pair(j2, _):
            j = 2 * j2 + 1
            step(j, 1)

            @pl.when(j + 1 < n)
            def _():
                step(j + 1, 0)

            return 0

        lax.fori_loop(0, n // 2, pair, 0)
        gwait(1)
        scatter(1)
        plsc.subcore_barrier()
        for j in range(RPT // 64):
            r0 = s * RPT + j * 64
            pltpu.sync_copy(acc.at[pl.ds(r0, 64)], zbuf)
            pltpu.sync_copy(zbuf, out_hbm.at[c, pl.ds(r0, 64)])

    return agg


_BN = 1000
_GRID = (N // _BN,)


def _tc_prep_body(x_ref, w_ref, cnt_ref, ycat_ref, d_ref):
    deg = cnt_ref[0, :, 0:1] + cnt_ref[1, :, 0:1] + 1.0
    dv = lax.rsqrt(deg)
    mm = jnp.dot(x_ref[...], w_ref[...], preferred_element_type=jnp.float32)
    y = mm * dv
    ycat_ref[0] = y[:, :IN_CH]
    ycat_ref[1] = y[:, IN_CH:]
    d_ref[...] = jnp.broadcast_to(dv, (_BN, IN_CH))


def _tc_prep(x, w1, cnt):
    return pl.pallas_call(
        _tc_prep_body,
        grid=_GRID,
        in_specs=[
            pl.BlockSpec((_BN, IN_CH), lambda i: (i, 0)),
            pl.BlockSpec((IN_CH, HID), lambda i: (0, 0)),
            pl.BlockSpec((NC, _BN, 16), lambda i: (0, i, 0)),
        ],
        out_specs=[
            pl.BlockSpec((NC, _BN, IN_CH), lambda i: (0, i, 0)),
            pl.BlockSpec((_BN, IN_CH), lambda i: (i, 0)),
        ],
        out_shape=[
            jax.ShapeDtypeStruct((NC, N, IN_CH), jnp.float32),
            jax.ShapeDtypeStruct((N, IN_CH), jnp.float32),
        ],
    )(x, w1, cnt)


def _tc_mid_body(agg_ref, y_ref, d_ref, b_ref, w_ref, out_ref):
    d = d_ref[...]
    h0 = jnp.maximum(d * (agg_ref[0] + y_ref[0]) + b_ref[0], 0.0)
    h1 = jnp.maximum(d * (agg_ref[1] + y_ref[1]) + b_ref[1], 0.0)
    h = jnp.concatenate([h0, h1], axis=1)
    mm = jnp.dot(h, w_ref[...], preferred_element_type=jnp.float32)
    out_ref[...] = mm * d


def _tc_mid(agg1, ycat1, d, b1r, w2):
    return pl.pallas_call(
        _tc_mid_body,
        grid=_GRID,
        in_specs=[
            pl.BlockSpec((NC, _BN, IN_CH), lambda i: (0, i, 0)),
            pl.BlockSpec((NC, _BN, IN_CH), lambda i: (0, i, 0)),
            pl.BlockSpec((_BN, IN_CH), lambda i: (i, 0)),
            pl.BlockSpec((NC, 1, IN_CH), lambda i: (0, 0, 0)),
            pl.BlockSpec((HID, OUT_CH), lambda i: (0, 0)),
        ],
        out_specs=pl.BlockSpec((_BN, OUT_CH), lambda i: (i, 0)),
        out_shape=jax.ShapeDtypeStruct((N, OUT_CH), jnp.float32),
    )(agg1, ycat1, d, b1r, w2)


def _tc_out_body(agg_ref, y_ref, d_ref, b_ref, out_ref):
    d = d_ref[...]
    s = agg_ref[0] + agg_ref[1] + y_ref[...]
    out_ref[...] = jnp.maximum(d * s + b_ref[...], 0.0)


def _tc_out(agg2, y2, d, b2r):
    return pl.pallas_call(
        _tc_out_body,
        grid=_GRID,
        in_specs=[
            pl.BlockSpec((NC, _BN, OUT_CH), lambda i: (0, i, 0)),
            pl.BlockSpec((_BN, OUT_CH), lambda i: (i, 0)),
            pl.BlockSpec((_BN, IN_CH), lambda i: (i, 0)),
            pl.BlockSpec((1, OUT_CH), lambda i: (0, 0)),
        ],
        out_specs=pl.BlockSpec((_BN, OUT_CH), lambda i: (i, 0)),
        out_shape=jax.ShapeDtypeStruct((N, OUT_CH), jnp.float32),
    )(agg2, y2, d, b2r)


@jax.jit
def kernel(x, edge_index, W1, b1, W2, b2):
    src = jnp.concatenate(
        [edge_index[0], jnp.zeros((EPAD - E,), jnp.int32)])
    dst = jnp.concatenate(
        [edge_index[1], jnp.full((EPAD - E,), N, jnp.int32)])
    cnt = _make_deg_kernel()(dst.reshape(EROWS, CH))
    ycat1, d = _tc_prep(x, W1, cnt)
    agg1 = _make_agg_kernel(IN_CH, True)(
        ycat1.reshape(NC * N, IN_CH), src, dst)
    y2 = _tc_mid(agg1, ycat1, d, b1.reshape(NC, 1, IN_CH), W2)
    agg2 = _make_agg_kernel(OUT_CH, False)(y2, src, dst)
    return _tc_out(agg2, y2, d, b2.reshape(1, OUT_CH))


# speedup vs baseline: 1.1302x; 1.1302x over previous
import functools

import jax
import jax.numpy as jnp
from jax import lax
from jax.experimental import pallas as pl
from jax.experimental.pallas import tpu as pltpu
from jax.experimental.pallas import tpu_sc as plsc

N = 10000
E = 320000
IN_CH = 128
HID = 256
OUT_CH = 128

NC = 2
NS = 16
CH = 80
EROWS = 4096
EPAD = EROWS * CH
NPAD = 10240
ZR = 128
RPT = NPAD // NS

_PIPE = True
_D = 1
_IP = 2

_MESH = dict(core_axis_name="c", subcore_axis_name="s", num_cores=NC,
             num_subcores=NS)


def _fill(ref, nrows, ncols, value):
    vec = jnp.full((16,), value, jnp.float32)

    def body(i, _):
        for t in range(ncols // 16):
            ref[i, pl.ds(t * 16, 16)] = vec
        return 0

    lax.fori_loop(0, nrows, body, 0)


_DROWS = EROWS // (NC * NS)
CHD = 128
EROWSD = EPAD // CHD
_DROWSD = EROWSD // (NC * NS)


@functools.cache
def _make_deg_kernel():
    @functools.partial(
        pl.kernel,
        out_type=jax.ShapeDtypeStruct((NC, NPAD, 16), jnp.float32),
        mesh=plsc.VectorSubcoreMesh(**_MESH),
        scratch_types=[
            pltpu.VMEM((_DROWSD, CHD), jnp.int32),
            pltpu.VMEM((CHD, 16), jnp.float32),
            pltpu.VMEM((ZR, 16), jnp.float32),
            pltpu.VMEM_SHARED((NPAD, 16), jnp.float32),
            pltpu.SemaphoreType.DMA,
            pltpu.SemaphoreType.DMA,
        ],
    )
    def deg(dst_hbm, out_hbm, didx, ones, zbuf, acc, sem0, sem1):
        c = lax.axis_index("c")
        s = lax.axis_index("s")
        w32 = c * NS + s
        base = w32 * _DROWSD
        n = _DROWSD

        slab = pltpu.async_copy(dst_hbm.at[pl.ds(base, _DROWSD)], didx, sem0)
        _fill(ones, CHD, 16, 1.0)
        _fill(zbuf, ZR, 16, 0.0)
        for j in range(RPT // ZR):
            pltpu.sync_copy(zbuf, acc.at[pl.ds(s * RPT + j * ZR, ZR)])
        slab.wait()
        plsc.subcore_barrier()

        def sadd(jj, sem):
            return pltpu.async_copy(ones, acc.at[didx.at[jj]], sem, add=True)

        sadd(0, sem0)
        sadd(1, sem1)

        def pair(j2, _):
            jj = 2 * j2
            pltpu.make_async_copy(ones, acc.at[didx.at[jj]], sem0).wait()

            @pl.when(jj + 2 < n)
            def _():
                sadd(jj + 2, sem0)

            pltpu.make_async_copy(ones, acc.at[didx.at[jj]], sem1).wait()

            @pl.when(jj + 3 < n)
            def _():
                sadd(jj + 3, sem1)

            return 0

        lax.fori_loop(0, n // 2, pair, 0)
        plsc.subcore_barrier()
        for j in range(RPT // ZR):
            r0 = s * RPT + j * ZR
            pltpu.sync_copy(acc.at[pl.ds(r0, ZR)], zbuf)
            pltpu.sync_copy(zbuf, out_hbm.at[c, pl.ds(r0, ZR)])

    return deg


@functools.cache
def _make_agg_kernel(w, split_channels):
    if split_channels:
        n, stride = EROWS // NS, NS
    else:
        n, stride = _DROWS, NC * NS
    nwb = RPT // CH

    @functools.partial(
        pl.kernel,
        out_type=jax.ShapeDtypeStruct((NC, NPAD, w), jnp.float32),
        mesh=plsc.VectorSubcoreMesh(**_MESH),
        scratch_types=[
            [pltpu.VMEM((CH,), jnp.int32)] * 8,
            [pltpu.VMEM((CH,), jnp.int32)] * 8,
            pltpu.VMEM((CH, w), jnp.float32),
            pltpu.VMEM((CH, w), jnp.float32),
            pltpu.VMEM((CH, w), jnp.float32),
            pltpu.VMEM((CH, w), jnp.float32),
            pltpu.VMEM_SHARED((NPAD, w), jnp.float32),
            [pltpu.SemaphoreType.DMA] * 4,
            [pltpu.SemaphoreType.DMA] * 4,
            [pltpu.SemaphoreType.DMA] * 4,
        ],
    )
    def agg(y_hbm, src_hbm, dst_hbm, out_hbm, sidx, didx, r0, r1, r2, r3,
            acc, isems, gsems, ssems):
        c = lax.axis_index("c")
        s = lax.axis_index("s")
        tbase = s if split_channels else c * NS + s

        rows = (r0, r1, r2, r3)

        def idx_start(j, b8):
            e0 = (tbase + j * stride) * CH
            pltpu.async_copy(src_hbm.at[pl.ds(e0, CH)], sidx[b8],
                             isems[b8 % 4])
            pltpu.async_copy(dst_hbm.at[pl.ds(e0, CH)], didx[b8],
                             isems[b8 % 4])

        def idx_wait(b8):
            pltpu.make_async_copy(src_hbm.at[pl.ds(0, CH)], sidx[b8],
                                  isems[b8 % 4]).wait()
            pltpu.make_async_copy(src_hbm.at[pl.ds(0, CH)], didx[b8],
                                  isems[b8 % 4]).wait()

        def shift_src(b8):
            if split_channels:
                @pl.when(c == 1)
                def _():
                    for t in range(CH // 16):
                        sl = pl.ds(t * 16, 16)
                        sidx[b8][sl] = sidx[b8][sl] + N

        def gstart(b8, b4):
            pltpu.async_copy(y_hbm.at[sidx[b8]], rows[b4], gsems[b4])

        def gwait(b8, b4):
            pltpu.make_async_copy(y_hbm.at[sidx[b8]], rows[b4],
                                  gsems[b4]).wait()

        def sstart(b8, b4):
            pltpu.async_copy(rows[b4], acc.at[didx[b8]], ssems[b4],
                             add=True)

        def swait(b8, b4):
            pltpu.make_async_copy(rows[b4], acc.at[didx[b8]],
                                  ssems[b4]).wait()

        if _PIPE:
            for k in range(_IP):
                idx_start(k, k)
        for b in range(4):
            _fill(rows[b], CH, w, 0.0)
        for k in range(nwb):
            pltpu.async_copy(rows[k % 4],
                             acc.at[pl.ds(s * RPT + k * CH, CH)],
                             gsems[k % 4])
        for k in range(nwb):
            pltpu.make_async_copy(rows[k % 4],
                                  acc.at[pl.ds(s * RPT + k * CH, CH)],
                                  gsems[k % 4]).wait()
        plsc.subcore_barrier()

        def chunk_sync(j, _):
            idx_start(j, 0)
            idx_wait(0)
            shift_src(0)
            gstart(0, 0)
            gwait(0, 0)
            sstart(0, 0)
            swait(0, 0)
            return 0

        if not _PIPE:
            lax.fori_loop(0, n, chunk_sync, 0)
            plsc.subcore_barrier()
            for k in range(nwb):
                blk = pl.ds(s * RPT + k * CH, CH)
                pltpu.sync_copy(acc.at[blk], rows[k % 4])
                pltpu.sync_copy(rows[k % 4], out_hbm.at[c, blk])
            return

        def position(j, r, first, last):
            idx_wait(r)
            shift_src(r)
            if not (first and j < _D + 1):
                swait((r - _D - 1) % 8, (r - _D - 1) % 4)
            gstart(r, r % 4)
            if not (first and j < _D):
                gwait((r - _D) % 8, (r - _D) % 4)
                sstart((r - _D) % 8, (r - _D) % 4)
            if not (last and j + _IP >= n):
                idx_start(j + _IP, (r + _IP) % 8)

        for r in range(8):
            position(r, r, True, False)

        def oct(q, _):
            for r in range(8):
                position(8 * q + r, r, False, False)
            return 0

        lax.fori_loop(1, n // 8 - 1, oct, 0)
        for r in range(8):
            position(n - 8 + r, r, False, True)
        for k in range(_D):
            j = n - _D + k
            gwait(j % 8, j % 4)
            sstart(j % 8, j % 4)
        for k in range(_D + 1):
            j = n - _D - 1 + k
            swait(j % 8, j % 4)
        plsc.subcore_barrier()
        for k in range(nwb):
            blk = pl.ds(s * RPT + k * CH, CH)
            pltpu.sync_copy(acc.at[blk], rows[k % 4])
            pltpu.sync_copy(rows[k % 4], out_hbm.at[c, blk])

    return agg


_BN = 1000
_GRID = (N // _BN,)


def _tc_prep_body(x_ref, w_ref, cnt_ref, ycat_ref, d_ref):
    deg = cnt_ref[0, :, 0:1] + cnt_ref[1, :, 0:1] + 1.0
    dv = lax.rsqrt(deg)
    mm = jnp.dot(x_ref[...], w_ref[...], preferred_element_type=jnp.float32)
    y = mm * dv
    ycat_ref[0] = y[:, :IN_CH]
    ycat_ref[1] = y[:, IN_CH:]
    d_ref[...] = jnp.broadcast_to(dv, (_BN, IN_CH))


def _tc_prep(x, w1, cnt):
    return pl.pallas_call(
        _tc_prep_body,
        grid=_GRID,
        in_specs=[
            pl.BlockSpec((_BN, IN_CH), lambda i: (i, 0)),
            pl.BlockSpec((IN_CH, HID), lambda i: (0, 0)),
            pl.BlockSpec((NC, _BN, 16), lambda i: (0, i, 0)),
        ],
        out_specs=[
            pl.BlockSpec((NC, _BN, IN_CH), lambda i: (0, i, 0)),
            pl.BlockSpec((_BN, IN_CH), lambda i: (i, 0)),
        ],
        out_shape=[
            jax.ShapeDtypeStruct((NC, N, IN_CH), jnp.float32),
            jax.ShapeDtypeStruct((N, IN_CH), jnp.float32),
        ],
    )(x, w1, cnt)


def _tc_mid_body(agg_ref, y_ref, d_ref, b_ref, w_ref, out_ref):
    d = d_ref[...]
    h0 = jnp.maximum(d * (agg_ref[0] + y_ref[0]) + b_ref[0], 0.0)
    h1 = jnp.maximum(d * (agg_ref[1] + y_ref[1]) + b_ref[1], 0.0)
    h = jnp.concatenate([h0, h1], axis=1)
    mm = jnp.dot(h, w_ref[...], preferred_element_type=jnp.float32)
    out_ref[...] = mm * d


def _tc_mid(agg1, ycat1, d, b1r, w2):
    return pl.pallas_call(
        _tc_mid_body,
        grid=_GRID,
        in_specs=[
            pl.BlockSpec((NC, _BN, IN_CH), lambda i: (0, i, 0)),
            pl.BlockSpec((NC, _BN, IN_CH), lambda i: (0, i, 0)),
            pl.BlockSpec((_BN, IN_CH), lambda i: (i, 0)),
            pl.BlockSpec((NC, 1, IN_CH), lambda i: (0, 0, 0)),
            pl.BlockSpec((HID, OUT_CH), lambda i: (0, 0)),
        ],
        out_specs=pl.BlockSpec((_BN, OUT_CH), lambda i: (i, 0)),
        out_shape=jax.ShapeDtypeStruct((N, OUT_CH), jnp.float32),
    )(agg1, ycat1, d, b1r, w2)


def _tc_out_body(agg_ref, y_ref, d_ref, b_ref, out_ref):
    d = d_ref[...]
    s = agg_ref[0] + agg_ref[1] + y_ref[...]
    out_ref[...] = jnp.maximum(d * s + b_ref[...], 0.0)


def _tc_out(agg2, y2, d, b2r):
    return pl.pallas_call(
        _tc_out_body,
        grid=_GRID,
        in_specs=[
            pl.BlockSpec((NC, _BN, OUT_CH), lambda i: (0, i, 0)),
            pl.BlockSpec((_BN, OUT_CH), lambda i: (i, 0)),
            pl.BlockSpec((_BN, IN_CH), lambda i: (i, 0)),
            pl.BlockSpec((1, OUT_CH), lambda i: (0, 0)),
        ],
        out_specs=pl.BlockSpec((_BN, OUT_CH), lambda i: (i, 0)),
        out_shape=jax.ShapeDtypeStruct((N, OUT_CH), jnp.float32),
    )(agg2, y2, d, b2r)


@jax.jit
def kernel(x, edge_index, W1, b1, W2, b2):
    src = jnp.concatenate(
        [edge_index[0], jnp.zeros((EPAD - E,), jnp.int32)])
    dst = jnp.concatenate(
        [edge_index[1], jnp.full((EPAD - E,), N, jnp.int32)])
    cnt = _make_deg_kernel()(dst.reshape(EROWSD, CHD))
    ycat1, d = _tc_prep(x, W1, cnt)
    agg1 = _make_agg_kernel(IN_CH, True)(
        ycat1.reshape(NC * N, IN_CH), src, dst)
    y2 = _tc_mid(agg1, ycat1, d, b1.reshape(NC, 1, IN_CH), W2)
    agg2 = _make_agg_kernel(OUT_CH, False)(y2, src, dst)
    return _tc_out(agg2, y2, d, b2.reshape(1, OUT_CH))


# speedup vs baseline: 1.1753x; 1.0399x over previous
import functools

import jax
import jax.numpy as jnp
from jax import lax
from jax.experimental import pallas as pl
from jax.experimental.pallas import tpu as pltpu
from jax.experimental.pallas import tpu_sc as plsc

N = 10000
E = 320000
IN_CH = 128
HID = 256
OUT_CH = 128

NC = 2
NS = 16
CH = 128
EROWS = 2560
EPAD = EROWS * CH
NPAD = 10240
ZR = 128
RPT = NPAD // NS

_PIPE = True
_D = 1
_IP = 2

_MESH = dict(core_axis_name="c", subcore_axis_name="s", num_cores=NC,
             num_subcores=NS)


def _fill(ref, nrows, ncols, value):
    vec = jnp.full((16,), value, jnp.float32)

    def body(i, _):
        for t in range(ncols // 16):
            ref[i, pl.ds(t * 16, 16)] = vec
        return 0

    lax.fori_loop(0, nrows, body, 0)


_DROWS = EROWS // (NC * NS)
CHD = 128
EROWSD = EPAD // CHD
_DROWSD = EROWSD // (NC * NS)


@functools.cache
def _make_deg_kernel():
    @functools.partial(
        pl.kernel,
        out_type=jax.ShapeDtypeStruct((NC, NPAD, 16), jnp.float32),
        mesh=plsc.VectorSubcoreMesh(**_MESH),
        scratch_types=[
            pltpu.VMEM((_DROWSD, CHD), jnp.int32),
            pltpu.VMEM((CHD, 16), jnp.float32),
            pltpu.VMEM((ZR, 16), jnp.float32),
            pltpu.VMEM_SHARED((NPAD, 16), jnp.float32),
            pltpu.SemaphoreType.DMA,
            pltpu.SemaphoreType.DMA,
        ],
    )
    def deg(dst_hbm, out_hbm, didx, ones, zbuf, acc, sem0, sem1):
        c = lax.axis_index("c")
        s = lax.axis_index("s")
        w32 = c * NS + s
        base = w32 * _DROWSD
        n = _DROWSD

        slab = pltpu.async_copy(dst_hbm.at[pl.ds(base, _DROWSD)], didx, sem0)
        _fill(ones, CHD, 16, 1.0)
        _fill(zbuf, ZR, 16, 0.0)
        for j in range(RPT // ZR):
            pltpu.sync_copy(zbuf, acc.at[pl.ds(s * RPT + j * ZR, ZR)])
        slab.wait()
        plsc.subcore_barrier()

        def sadd(jj, sem):
            return pltpu.async_copy(ones, acc.at[didx.at[jj]], sem, add=True)

        sadd(0, sem0)
        sadd(1, sem1)

        def pair(j2, _):
            jj = 2 * j2
            pltpu.make_async_copy(ones, acc.at[didx.at[jj]], sem0).wait()

            @pl.when(jj + 2 < n)
            def _():
                sadd(jj + 2, sem0)

            pltpu.make_async_copy(ones, acc.at[didx.at[jj]], sem1).wait()

            @pl.when(jj + 3 < n)
            def _():
                sadd(jj + 3, sem1)

            return 0

        lax.fori_loop(0, n // 2, pair, 0)
        plsc.subcore_barrier()
        for j in range(RPT // ZR):
            r0 = s * RPT + j * ZR
            pltpu.sync_copy(acc.at[pl.ds(r0, ZR)], zbuf)
            pltpu.sync_copy(zbuf, out_hbm.at[c, pl.ds(r0, ZR)])

    return deg


@functools.cache
def _make_agg_kernel(w, split_channels):
    if split_channels:
        n, stride = EROWS // NS, NS
    else:
        n, stride = _DROWS, NC * NS
    nwb = RPT // CH

    @functools.partial(
        pl.kernel,
        out_type=jax.ShapeDtypeStruct((NC, NPAD, w), jnp.float32),
        mesh=plsc.VectorSubcoreMesh(**_MESH),
        scratch_types=[
            [pltpu.VMEM((CH,), jnp.int32)] * 8,
            [pltpu.VMEM((CH,), jnp.int32)] * 8,
            pltpu.VMEM((CH, w), jnp.float32),
            pltpu.VMEM((CH, w), jnp.float32),
            pltpu.VMEM_SHARED((NPAD, w), jnp.float32),
            [pltpu.SemaphoreType.DMA] * 4,
            [pltpu.SemaphoreType.DMA] * 2,
            [pltpu.SemaphoreType.DMA] * 2,
        ],
    )
    def agg(y_hbm, src_hbm, dst_hbm, out_hbm, sidx, didx, r0, r1,
            acc, isems, gsems, ssems):
        c = lax.axis_index("c")
        s = lax.axis_index("s")
        tbase = s if split_channels else c * NS + s

        rows = (r0, r1)
        NB = len(rows)

        def idx_start(j, b8):
            e0 = (tbase + j * stride) * CH
            pltpu.async_copy(src_hbm.at[pl.ds(e0, CH)], sidx[b8],
                             isems[b8 % 4])
            pltpu.async_copy(dst_hbm.at[pl.ds(e0, CH)], didx[b8],
                             isems[b8 % 4])

        def idx_wait(b8):
            pltpu.make_async_copy(src_hbm.at[pl.ds(0, CH)], sidx[b8],
                                  isems[b8 % 4]).wait()
            pltpu.make_async_copy(src_hbm.at[pl.ds(0, CH)], didx[b8],
                                  isems[b8 % 4]).wait()

        def shift_src(b8):
            if split_channels:
                @pl.when(c == 1)
                def _():
                    for t in range(CH // 16):
                        sl = pl.ds(t * 16, 16)
                        sidx[b8][sl] = sidx[b8][sl] + N

        def gstart(b8, b4):
            pltpu.async_copy(y_hbm.at[sidx[b8]], rows[b4], gsems[b4])

        def gwait(b8, b4):
            pltpu.make_async_copy(y_hbm.at[sidx[b8]], rows[b4],
                                  gsems[b4]).wait()

        def sstart(b8, b4):
            pltpu.async_copy(rows[b4], acc.at[didx[b8]], ssems[b4],
                             add=True)

        def swait(b8, b4):
            pltpu.make_async_copy(rows[b4], acc.at[didx[b8]],
                                  ssems[b4]).wait()

        if _PIPE:
            for k in range(_IP):
                idx_start(k, k)
        for b in range(NB):
            _fill(rows[b], CH, w, 0.0)
        for k in range(nwb):
            pltpu.async_copy(rows[k % NB],
                             acc.at[pl.ds(s * RPT + k * CH, CH)],
                             gsems[k % NB])
        for k in range(nwb):
            pltpu.make_async_copy(rows[k % NB],
                                  acc.at[pl.ds(s * RPT + k * CH, CH)],
                                  gsems[k % NB]).wait()
        plsc.subcore_barrier()

        def chunk_sync(j, _):
            idx_start(j, 0)
            idx_wait(0)
            shift_src(0)
            gstart(0, 0)
            gwait(0, 0)
            sstart(0, 0)
            swait(0, 0)
            return 0

        if not _PIPE:
            lax.fori_loop(0, n, chunk_sync, 0)
            plsc.subcore_barrier()
            for k in range(nwb):
                blk = pl.ds(s * RPT + k * CH, CH)
                pltpu.sync_copy(acc.at[blk], rows[k % NB])
                pltpu.sync_copy(rows[k % NB], out_hbm.at[c, blk])
            return

        def position(j, r, first, last):
            idx_wait(r)
            shift_src(r)
            if not (first and j < _D + 1):
                swait((r - _D - 1) % 8, (r - _D - 1) % NB)
            gstart(r, r % NB)
            if not (first and j < _D):
                gwait((r - _D) % 8, (r - _D) % NB)
                sstart((r - _D) % 8, (r - _D) % NB)
            if not (last and j + _IP >= n):
                idx_start(j + _IP, (r + _IP) % 8)

        for r in range(8):
            position(r, r, True, False)

        def oct(q, _):
            for r in range(8):
                position(8 * q + r, r, False, False)
            return 0

        lax.fori_loop(1, n // 8 - 1, oct, 0)
        for r in range(8):
            position(n - 8 + r, r, False, True)
        for k in range(_D):
            j = n - _D + k
            gwait(j % 8, j % NB)
            sstart(j % 8, j % NB)
        for k in range(_D + 1):
            j = n - _D - 1 + k
            swait(j % 8, j % NB)
        plsc.subcore_barrier()
        for k in range(nwb):
            blk = pl.ds(s * RPT + k * CH, CH)
            pltpu.sync_copy(acc.at[blk], rows[k % NB])
            pltpu.sync_copy(rows[k % NB], out_hbm.at[c, blk])

    return agg


_BN = 1000
_GRID = (N // _BN,)


def _tc_prep_body(x_ref, w_ref, cnt_ref, ycat_ref, d_ref):
    deg = cnt_ref[0, :, 0:1] + cnt_ref[1, :, 0:1] + 1.0
    dv = lax.rsqrt(deg)
    mm = jnp.dot(x_ref[...], w_ref[...], preferred_element_type=jnp.float32)
    y = mm * dv
    ycat_ref[0] = y[:, :IN_CH]
    ycat_ref[1] = y[:, IN_CH:]
    d_ref[...] = jnp.broadcast_to(dv, (_BN, IN_CH))


def _tc_prep(x, w1, cnt):
    return pl.pallas_call(
        _tc_prep_body,
        grid=_GRID,
        in_specs=[
            pl.BlockSpec((_BN, IN_CH), lambda i: (i, 0)),
            pl.BlockSpec((IN_CH, HID), lambda i: (0, 0)),
            pl.BlockSpec((NC, _BN, 16), lambda i: (0, i, 0)),
        ],
        out_specs=[
            pl.BlockSpec((NC, _BN, IN_CH), lambda i: (0, i, 0)),
            pl.BlockSpec((_BN, IN_CH), lambda i: (i, 0)),
        ],
        out_shape=[
            jax.ShapeDtypeStruct((NC, N, IN_CH), jnp.float32),
            jax.ShapeDtypeStruct((N, IN_CH), jnp.float32),
        ],
    )(x, w1, cnt)


def _tc_mid_body(agg_ref, y_ref, d_ref, b_ref, w_ref, out_ref):
    d = d_ref[...]
    h0 = jnp.maximum(d * (agg_ref[0] + y_ref[0]) + b_ref[0], 0.0)
    h1 = jnp.maximum(d * (agg_ref[1] + y_ref[1]) + b_ref[1], 0.0)
    h = jnp.concatenate([h0, h1], axis=1)
    mm = jnp.dot(h, w_ref[...], preferred_element_type=jnp.float32)
    out_ref[...] = mm * d


def _tc_mid(agg1, ycat1, d, b1r, w2):
    return pl.pallas_call(
        _tc_mid_body,
        grid=_GRID,
        in_specs=[
            pl.BlockSpec((NC, _BN, IN_CH), lambda i: (0, i, 0)),
            pl.BlockSpec((NC, _BN, IN_CH), lambda i: (0, i, 0)),
            pl.BlockSpec((_BN, IN_CH), lambda i: (i, 0)),
            pl.BlockSpec((NC, 1, IN_CH), lambda i: (0, 0, 0)),
            pl.BlockSpec((HID, OUT_CH), lambda i: (0, 0)),
        ],
        out_specs=pl.BlockSpec((_BN, OUT_CH), lambda i: (i, 0)),
        out_shape=jax.ShapeDtypeStruct((N, OUT_CH), jnp.float32),
    )(agg1, ycat1, d, b1r, w2)


def _tc_out_body(agg_ref, y_ref, d_ref, b_ref, out_ref):
    d = d_ref[...]
    s = agg_ref[0] + agg_ref[1] + y_ref[...]
    out_ref[...] = jnp.maximum(d * s + b_ref[...], 0.0)


def _tc_out(agg2, y2, d, b2r):
    return pl.pallas_call(
        _tc_out_body,
        grid=_GRID,
        in_specs=[
            pl.BlockSpec((NC, _BN, OUT_CH), lambda i: (0, i, 0)),
            pl.BlockSpec((_BN, OUT_CH), lambda i: (i, 0)),
            pl.BlockSpec((_BN, IN_CH), lambda i: (i, 0)),
            pl.BlockSpec((1, OUT_CH), lambda i: (0, 0)),
        ],
        out_specs=pl.BlockSpec((_BN, OUT_CH), lambda i: (i, 0)),
        out_shape=jax.ShapeDtypeStruct((N, OUT_CH), jnp.float32),
    )(agg2, y2, d, b2r)


@jax.jit
def kernel(x, edge_index, W1, b1, W2, b2):
    src = jnp.concatenate(
        [edge_index[0], jnp.zeros((EPAD - E,), jnp.int32)])
    dst = jnp.concatenate(
        [edge_index[1], jnp.full((EPAD - E,), N, jnp.int32)])
    cnt = _make_deg_kernel()(dst.reshape(EROWSD, CHD))
    ycat1, d = _tc_prep(x, W1, cnt)
    agg1 = _make_agg_kernel(IN_CH, True)(
        ycat1.reshape(NC * N, IN_CH), src, dst)
    y2 = _tc_mid(agg1, ycat1, d, b1.reshape(NC, 1, IN_CH), W2)
    agg2 = _make_agg_kernel(OUT_CH, False)(y2, src, dst)
    return _tc_out(agg2, y2, d, b2.reshape(1, OUT_CH))


# speedup vs baseline: 1.3613x; 1.1583x over previous
import functools

import jax
import jax.numpy as jnp
from jax import lax
from jax.experimental import pallas as pl
from jax.experimental.pallas import tpu as pltpu
from jax.experimental.pallas import tpu_sc as plsc

N = 10000
E = 320000
IN_CH = 128
HID = 256
OUT_CH = 128

NC = 2
NS = 16
CH = 128
EROWS = 2560
EPAD = EROWS * CH
NPAD = 10240
ZR = 128
RPT = NPAD // NS

_D = 1
_IP = 2

_MESH = dict(core_axis_name="c", subcore_axis_name="s", num_cores=NC,
             num_subcores=NS)


def _fill(ref, nrows, ncols, value):
    vec = jnp.full((16,), value, jnp.float32)

    def body(i, _):
        for t in range(ncols // 16):
            ref[i, pl.ds(t * 16, 16)] = vec
        return 0

    lax.fori_loop(0, nrows, body, 0)


_DROWS = EROWS // (NC * NS)
CHD = 128
EROWSD = EPAD // CHD
_DROWSD = EROWSD // (NC * NS)


@functools.cache
def _make_deg_kernel():
    @functools.partial(
        pl.kernel,
        out_type=jax.ShapeDtypeStruct((NC, NPAD, 16), jnp.float32),
        mesh=plsc.VectorSubcoreMesh(**_MESH),
        scratch_types=[
            pltpu.VMEM((_DROWSD, CHD), jnp.int32),
            pltpu.VMEM((CHD, 16), jnp.float32),
            pltpu.VMEM((ZR, 16), jnp.float32),
            pltpu.VMEM_SHARED((NPAD, 16), jnp.float32),
            pltpu.SemaphoreType.DMA,
            pltpu.SemaphoreType.DMA,
        ],
    )
    def deg(dst_hbm, out_hbm, didx, ones, zbuf, acc, sem0, sem1):
        c = lax.axis_index("c")
        s = lax.axis_index("s")
        w32 = c * NS + s
        base = w32 * _DROWSD
        n = _DROWSD

        slab = pltpu.async_copy(dst_hbm.at[pl.ds(base, _DROWSD)], didx, sem0)
        _fill(ones, CHD, 16, 1.0)
        _fill(zbuf, ZR, 16, 0.0)
        for j in range(RPT // ZR):
            pltpu.sync_copy(zbuf, acc.at[pl.ds(s * RPT + j * ZR, ZR)])
        slab.wait()
        plsc.subcore_barrier()

        def sadd(jj, sem):
            return pltpu.async_copy(ones, acc.at[didx.at[jj]], sem, add=True)

        sadd(0, sem0)
        sadd(1, sem1)

        def pair(j2, _):
            jj = 2 * j2
            pltpu.make_async_copy(ones, acc.at[didx.at[jj]], sem0).wait()

            @pl.when(jj + 2 < n)
            def _():
                sadd(jj + 2, sem0)

            pltpu.make_async_copy(ones, acc.at[didx.at[jj]], sem1).wait()

            @pl.when(jj + 3 < n)
            def _():
                sadd(jj + 3, sem1)

            return 0

        lax.fori_loop(0, n // 2, pair, 0)
        plsc.subcore_barrier()
        for j in range(RPT // ZR):
            r0 = s * RPT + j * ZR
            pltpu.sync_copy(acc.at[pl.ds(r0, ZR)], zbuf)
            pltpu.sync_copy(zbuf, out_hbm.at[c, pl.ds(r0, ZR)])

    return deg


@functools.cache
def _make_agg_kernel(w, split_channels, dup=False):
    if split_channels:
        n, stride = EROWS // NS, NS
    else:
        n, stride = _DROWS, NC * NS
    shift = split_channels or dup
    nwb = RPT // CH

    @functools.partial(
        pl.kernel,
        out_type=jax.ShapeDtypeStruct((NC, NPAD, w), jnp.float32),
        mesh=plsc.VectorSubcoreMesh(**_MESH),
        scratch_types=[
            [pltpu.VMEM((CH,), jnp.int32)] * 8,
            [pltpu.VMEM((CH,), jnp.int32)] * 8,
            pltpu.VMEM((CH, w), jnp.float32),
            pltpu.VMEM((CH, w), jnp.float32),
            pltpu.VMEM_SHARED((NPAD, w), jnp.float32),
            [pltpu.SemaphoreType.DMA] * 4,
            [pltpu.SemaphoreType.DMA] * 2,
            [pltpu.SemaphoreType.DMA] * 2,
        ],
    )
    def agg(y_hbm, src_hbm, dst_hbm, out_hbm, sidx, didx, r0, r1,
            acc, isems, gsems, ssems):
        c = lax.axis_index("c")
        s = lax.axis_index("s")
        tbase = s if split_channels else c * NS + s

        rows = (r0, r1)
        NB = len(rows)

        def idx_start(j, b8):
            e0 = (tbase + j * stride) * CH
            pltpu.async_copy(src_hbm.at[pl.ds(e0, CH)], sidx[b8],
                             isems[b8 % 4])
            pltpu.async_copy(dst_hbm.at[pl.ds(e0, CH)], didx[b8],
                             isems[b8 % 4])

        def idx_wait(b8):
            pltpu.make_async_copy(src_hbm.at[pl.ds(0, CH)], sidx[b8],
                                  isems[b8 % 4]).wait()
            pltpu.make_async_copy(src_hbm.at[pl.ds(0, CH)], didx[b8],
                                  isems[b8 % 4]).wait()

        def shift_src(b8):
            if shift:
                @pl.when(c == 1)
                def _():
                    for t in range(CH // 16):
                        sl = pl.ds(t * 16, 16)
                        sidx[b8][sl] = sidx[b8][sl] + N

        def gstart(b8, b4):
            pltpu.async_copy(y_hbm.at[sidx[b8]], rows[b4], gsems[b4])

        def gwait(b8, b4):
            pltpu.make_async_copy(y_hbm.at[sidx[b8]], rows[b4],
                                  gsems[b4]).wait()

        def sstart(b8, b4):
            pltpu.async_copy(rows[b4], acc.at[didx[b8]], ssems[b4],
                             add=True)

        def swait(b8, b4):
            pltpu.make_async_copy(rows[b4], acc.at[didx[b8]],
                                  ssems[b4]).wait()

        for k in range(_IP):
            idx_start(k, k)
        for b in range(NB):
            _fill(rows[b], CH, w, 0.0)
        for k in range(nwb):
            pltpu.async_copy(rows[k % NB],
                             acc.at[pl.ds(s * RPT + k * CH, CH)],
                             gsems[k % NB])
        for k in range(nwb):
            pltpu.make_async_copy(rows[k % NB],
                                  acc.at[pl.ds(s * RPT + k * CH, CH)],
                                  gsems[k % NB]).wait()
        plsc.subcore_barrier()

        def position(j, r, first, last):
            idx_wait(r)
            shift_src(r)
            if not (first and j < _D + 1):
                swait((r - _D - 1) % 8, (r - _D - 1) % NB)
            gstart(r, r % NB)
            if not (first and j < _D):
                gwait((r - _D) % 8, (r - _D) % NB)
                sstart((r - _D) % 8, (r - _D) % NB)
            if not (last and j + _IP >= n):
                idx_start(j + _IP, (r + _IP) % 8)

        for r in range(8):
            position(r, r, True, False)

        def oct(q, _):
            for r in range(8):
                position(8 * q + r, r, False, False)
            return 0

        lax.fori_loop(1, n // 8 - 1, oct, 0)
        for r in range(8):
            position(n - 8 + r, r, False, True)
        for k in range(_D):
            j = n - _D + k
            gwait(j % 8, j % NB)
            sstart(j % 8, j % NB)
        for k in range(_D + 1):
            j = n - _D - 1 + k
            swait(j % 8, j % NB)
        plsc.subcore_barrier()
        for k in range(nwb):
            blk = pl.ds(s * RPT + k * CH, CH)
            pltpu.sync_copy(acc.at[blk], rows[k % NB])
            pltpu.sync_copy(rows[k % NB], out_hbm.at[c, blk])

    return agg


_BN = 1000
_GRID = (N // _BN,)


def _tc_prep_body(x_ref, w_ref, cnt_ref, ycat_ref, d_ref):
    deg = cnt_ref[0, :, 0:1] + cnt_ref[1, :, 0:1] + 1.0
    dv = lax.rsqrt(deg)
    mm = jnp.dot(x_ref[...], w_ref[...], preferred_element_type=jnp.float32)
    y = mm * dv
    ycat_ref[0] = y[:, :IN_CH]
    ycat_ref[1] = y[:, IN_CH:]
    d_ref[...] = jnp.broadcast_to(dv, (_BN, IN_CH))


def _tc_prep(x, w1, cnt):
    return pl.pallas_call(
        _tc_prep_body,
        grid=_GRID,
        in_specs=[
            pl.BlockSpec((_BN, IN_CH), lambda i: (i, 0)),
            pl.BlockSpec((IN_CH, HID), lambda i: (0, 0)),
            pl.BlockSpec((NC, _BN, 16), lambda i: (0, i, 0)),
        ],
        out_specs=[
            pl.BlockSpec((NC, _BN, IN_CH), lambda i: (0, i, 0)),
            pl.BlockSpec((_BN, IN_CH), lambda i: (i, 0)),
        ],
        out_shape=[
            jax.ShapeDtypeStruct((NC, N, IN_CH), jnp.float32),
            jax.ShapeDtypeStruct((N, IN_CH), jnp.float32),
        ],
    )(x, w1, cnt)


def _tc_mid_body(agg_ref, y_ref, d_ref, b_ref, w_ref, out_ref):
    d = d_ref[...]
    h0 = jnp.maximum(d * (agg_ref[0] + y_ref[0]) + b_ref[0], 0.0)
    h1 = jnp.maximum(d * (agg_ref[1] + y_ref[1]) + b_ref[1], 0.0)
    h = jnp.concatenate([h0, h1], axis=1)
    mm = jnp.dot(h, w_ref[...], preferred_element_type=jnp.float32)
    y2 = mm * d
    out_ref[0] = y2
    out_ref[1] = y2


def _tc_mid(agg1, ycat1, d, b1r, w2):
    return pl.pallas_call(
        _tc_mid_body,
        grid=_GRID,
        in_specs=[
            pl.BlockSpec((NC, _BN, IN_CH), lambda i: (0, i, 0)),
            pl.BlockSpec((NC, _BN, IN_CH), lambda i: (0, i, 0)),
            pl.BlockSpec((_BN, IN_CH), lambda i: (i, 0)),
            pl.BlockSpec((NC, 1, IN_CH), lambda i: (0, 0, 0)),
            pl.BlockSpec((HID, OUT_CH), lambda i: (0, 0)),
        ],
        out_specs=pl.BlockSpec((NC, _BN, OUT_CH), lambda i: (0, i, 0)),
        out_shape=jax.ShapeDtypeStruct((NC, N, OUT_CH), jnp.float32),
    )(agg1, ycat1, d, b1r, w2)


def _tc_out_body(agg_ref, y_ref, d_ref, b_ref, out_ref):
    d = d_ref[...]
    s = agg_ref[0] + agg_ref[1] + y_ref[...]
    out_ref[...] = jnp.maximum(d * s + b_ref[...], 0.0)


def _tc_out(agg2, y2, d, b2r):
    return pl.pallas_call(
        _tc_out_body,
        grid=_GRID,
        in_specs=[
            pl.BlockSpec((NC, _BN, OUT_CH), lambda i: (0, i, 0)),
            pl.BlockSpec((_BN, OUT_CH), lambda i: (i, 0)),
            pl.BlockSpec((_BN, IN_CH), lambda i: (i, 0)),
            pl.BlockSpec((1, OUT_CH), lambda i: (0, 0)),
        ],
        out_specs=pl.BlockSpec((_BN, OUT_CH), lambda i: (i, 0)),
        out_shape=jax.ShapeDtypeStruct((N, OUT_CH), jnp.float32),
    )(agg2, y2, d, b2r)


@jax.jit
def kernel(x, edge_index, W1, b1, W2, b2):
    src = jnp.concatenate(
        [edge_index[0], jnp.zeros((EPAD - E,), jnp.int32)])
    dst = jnp.concatenate(
        [edge_index[1], jnp.full((EPAD - E,), N, jnp.int32)])
    cnt = _make_deg_kernel()(dst.reshape(EROWSD, CHD))
    ycat1, d = _tc_prep(x, W1, cnt)
    agg1 = _make_agg_kernel(IN_CH, True)(
        ycat1.reshape(NC * N, IN_CH), src, dst)
    y2cat = _tc_mid(agg1, ycat1, d, b1.reshape(NC, 1, IN_CH), W2)
    agg2 = _make_agg_kernel(OUT_CH, False, True)(
        y2cat.reshape(NC * N, OUT_CH), src, dst)
    return _tc_out(agg2, y2cat[0], d, b2.reshape(1, OUT_CH))
